# baseline (speedup 1.0000x reference)
import jax
import jax.numpy as jnp
from jax.experimental import pallas as pl
from jax.experimental.pallas import tpu as pltpu

_LAMBDA_CON = 0.4
_LAMBDA_BC = 0.2
_DX = 0.01
_DT = 0.001
_U0 = 1.0
_VISC = 0.01
_CROP = 512
_N = _CROP - 2
_B = 32


def _ddx(x):
    return x[1:-1, 1:-1] - x[1:-1, :-2]


def _ddy(x):
    return x[1:-1, 1:-1] - x[:-2, 1:-1]


def _loss_kernel(g_ref, pn_ref, out_ref):
    u = g_ref[0, 0]
    v = g_ref[0, 1]
    p = g_ref[0, 2]
    pn = pn_ref[0, 0]

    inv_dx = 1.0 / _DX
    c = _VISC / _DX

    y0 = (jnp.sum(u[0:1, 1:_N] + u[1:2, 1:_N], axis=1, keepdims=True)
          + jnp.sum(v[0:1, 1:_N + 1] + p[0:1, 1:_N + 1], axis=1,
                    keepdims=True))
    yl = (jnp.sum(2.0 * _U0 - u[_N:_N + 1, 1:_N] - u[_N + 1:_N + 2, 1:_N],
                  axis=1, keepdims=True)
          + jnp.sum(v[_N + 1:_N + 2, 1:_N + 1] + p[_N + 1:_N + 2, 1:_N + 1],
                    axis=1, keepdims=True))
    a = jnp.sum(v[1:_N, 0:8], axis=0, keepdims=True)
    b8 = jnp.sum(u[1:_N + 1, 0:8] + p[1:_N + 1, 0:8], axis=0,
                 keepdims=True)
    x0 = a[0:1, 0:1] + a[0:1, 1:2] + b8[0:1, 0:1]
    d = jnp.sum(v[1:_N, _CROP - 8:_CROP], axis=0, keepdims=True)
    c8 = jnp.sum(u[1:_N + 1, _CROP - 8:_CROP]
                 + p[1:_N + 1, _CROP - 8:_CROP], axis=0, keepdims=True)
    xl = d[0:1, 6:7] + d[0:1, 7:8] + c8[0:1, 7:8]
    bc = jnp.abs(y0) + jnp.abs(yl) + jnp.abs(x0) + jnp.abs(xl)

    cont = _ddx(u) + _ddy(v)
    cont_sum = jnp.sum(jnp.abs(cont), keepdims=True)

    ax_u = 0.5 * (u[:-1, :-1] + u[:-1, 1:])
    ax_v = 0.5 * (v[:-1, :-1] + v[:-1, 1:])
    ay_u = 0.5 * (u[:-1, :-1] + u[1:, :-1])
    ay_v = 0.5 * (v[:-1, :-1] + v[1:, :-1])

    fe = ax_u * ax_u - c * (u[:-1, 1:] - u[:-1, :-1])
    fn = ax_v * ay_u - c * (u[1:, :-1] - u[:-1, :-1])
    m_du = ((fe[1:, 1:-1] - fe[1:, :-2])
            + (fn[1:, 1:-1] - fn[:-1, 1:-1])
            + (p[1:-1, 2:-1] - p[1:-1, 1:-2]))

    fe2 = ay_u * ax_v - c * (v[:-1, 1:] - v[:-1, :-1])
    fn2 = ay_v * ay_v - c * (v[1:, :-1] - v[:-1, :-1])
    m_dv = ((fe2[1:-1, 1:] - fe2[1:-1, :-1])
            + (fn2[1:-1, 1:] - fn2[:-2, 1:])
            + (p[2:-1, 1:-1] - p[1:-2, 1:-1]))

    zc = jnp.zeros((_N, 1), jnp.float32)
    zr = jnp.zeros((1, _N), jnp.float32)
    ddx_m = (jnp.concatenate([m_du, zc], axis=1)
             - jnp.concatenate([zc, m_du], axis=1))
    ddy_m = (jnp.concatenate([m_dv, zr], axis=0)
             - jnp.concatenate([zr, m_dv], axis=0))

    pp = pn - p
    lapl = (4.0 * pp[1:-1, 1:-1] - pp[1:-1, 2:] - pp[1:-1, :-2]
            - pp[2:, 1:-1] - pp[:-2, 1:-1])
    pois = (lapl - ddx_m - ddy_m) + cont * (_DX / _DT)
    pois_sum = jnp.sum(jnp.abs(pois), keepdims=True)

    lam_res = 1.0 - _LAMBDA_CON - _LAMBDA_BC
    inv_mn = 1.0 / (_B * _N * _N)
    s_val = (_LAMBDA_CON * inv_mn * inv_dx) * cont_sum \
        + (lam_res * inv_mn / (_DX * _DX)) * pois_sum + _LAMBDA_BC * bc
    out_ref[0] = jnp.broadcast_to(s_val, (1, 128))


def kernel(gen_output, p_next_step):
    out = pl.pallas_call(
        _loss_kernel,
        grid=(_B,),
        in_specs=[
            pl.BlockSpec((1, 3, _CROP, _CROP), lambda b: (b, 0, 0, 0)),
            pl.BlockSpec((1, 1, _CROP, _CROP), lambda b: (b, 0, 0, 0)),
        ],
        out_specs=pl.BlockSpec((1, 1, 128), lambda b: (b, 0, 0)),
        out_shape=jax.ShapeDtypeStruct((_B, 1, 128), jnp.float32),
        compiler_params=pltpu.CompilerParams(
            dimension_semantics=("arbitrary",),
            vmem_limit_bytes=64 * 1024 * 1024,
        ),
    )(gen_output, p_next_step)
    return jnp.sum(out[:, 0, 0])


# speedup vs baseline: 1.0333x; 1.0333x over previous
import jax
import jax.numpy as jnp
from jax.experimental import pallas as pl
from jax.experimental.pallas import tpu as pltpu

_LAMBDA_CON = 0.4
_LAMBDA_BC = 0.2
_DX = 0.01
_DT = 0.001
_U0 = 1.0
_VISC = 0.01
_CROP = 512
_N = _CROP - 2
_B = 32


def _ddx(x):
    return x[1:-1, 1:-1] - x[1:-1, :-2]


def _ddy(x):
    return x[1:-1, 1:-1] - x[:-2, 1:-1]


def _loss_kernel(g_ref, pn_ref, out_ref):
    u = g_ref[0, 0]
    v = g_ref[0, 1]
    p = g_ref[0, 2]
    pn = pn_ref[0, 0]

    inv_dx = 1.0 / _DX
    c = _VISC / _DX

    y0 = (jnp.sum(u[0:1, 1:_N] + u[1:2, 1:_N], axis=1, keepdims=True)
          + jnp.sum(v[0:1, 1:_N + 1] + p[0:1, 1:_N + 1], axis=1,
                    keepdims=True))
    yl = (jnp.sum(2.0 * _U0 - u[_N:_N + 1, 1:_N] - u[_N + 1:_N + 2, 1:_N],
                  axis=1, keepdims=True)
          + jnp.sum(v[_N + 1:_N + 2, 1:_N + 1] + p[_N + 1:_N + 2, 1:_N + 1],
                    axis=1, keepdims=True))
    a = jnp.sum(v[1:_N, 0:8], axis=0, keepdims=True)
    b8 = jnp.sum(u[1:_N + 1, 0:8] + p[1:_N + 1, 0:8], axis=0,
                 keepdims=True)
    x0 = a[0:1, 0:1] + a[0:1, 1:2] + b8[0:1, 0:1]
    d = jnp.sum(v[1:_N, _CROP - 8:_CROP], axis=0, keepdims=True)
    c8 = jnp.sum(u[1:_N + 1, _CROP - 8:_CROP]
                 + p[1:_N + 1, _CROP - 8:_CROP], axis=0, keepdims=True)
    xl = d[0:1, 6:7] + d[0:1, 7:8] + c8[0:1, 7:8]
    bc = jnp.abs(y0) + jnp.abs(yl) + jnp.abs(x0) + jnp.abs(xl)

    cont = _ddx(u) + _ddy(v)
    cont_sum = jnp.sum(jnp.sum(jnp.abs(cont), axis=0, keepdims=True),
                       axis=1, keepdims=True)

    ax_u = 0.5 * (u[:-1, :-1] + u[:-1, 1:])
    ax_v = 0.5 * (v[:-1, :-1] + v[:-1, 1:])
    ay_u = 0.5 * (u[:-1, :-1] + u[1:, :-1])
    ay_v = 0.5 * (v[:-1, :-1] + v[1:, :-1])

    fe = ax_u * ax_u - c * (u[:-1, 1:] - u[:-1, :-1])
    fn = ax_v * ay_u - c * (u[1:, :-1] - u[:-1, :-1])
    m_du = ((fe[1:, 1:-1] - fe[1:, :-2])
            + (fn[1:, 1:-1] - fn[:-1, 1:-1])
            + (p[1:-1, 2:-1] - p[1:-1, 1:-2]))

    fe2 = ay_u * ax_v - c * (v[:-1, 1:] - v[:-1, :-1])
    fn2 = ay_v * ay_v - c * (v[1:, :-1] - v[:-1, :-1])
    m_dv = ((fe2[1:-1, 1:] - fe2[1:-1, :-1])
            + (fn2[1:-1, 1:] - fn2[:-2, 1:])
            + (p[2:-1, 1:-1] - p[1:-2, 1:-1]))

    zc = jnp.zeros((_N, 1), jnp.float32)
    zr = jnp.zeros((1, _N), jnp.float32)
    ddx_m = (jnp.concatenate([m_du, zc], axis=1)
             - jnp.concatenate([zc, m_du], axis=1))
    ddy_m = (jnp.concatenate([m_dv, zr], axis=0)
             - jnp.concatenate([zr, m_dv], axis=0))

    pp = pn - p
    lapl = (4.0 * pp[1:-1, 1:-1] - pp[1:-1, 2:] - pp[1:-1, :-2]
            - pp[2:, 1:-1] - pp[:-2, 1:-1])
    pois = (lapl - ddx_m - ddy_m) + cont * (_DX / _DT)
    pois_sum = jnp.sum(jnp.sum(jnp.abs(pois), axis=0, keepdims=True),
                       axis=1, keepdims=True)

    lam_res = 1.0 - _LAMBDA_CON - _LAMBDA_BC
    inv_mn = 1.0 / (_B * _N * _N)
    s_val = (_LAMBDA_CON * inv_mn * inv_dx) * cont_sum \
        + (lam_res * inv_mn / (_DX * _DX)) * pois_sum + _LAMBDA_BC * bc
    out_ref[0] = jnp.broadcast_to(s_val, (1, 128))


def kernel(gen_output, p_next_step):
    out = pl.pallas_call(
        _loss_kernel,
        grid=(_B,),
        in_specs=[
            pl.BlockSpec((1, 3, _CROP, _CROP), lambda b: (b, 0, 0, 0)),
            pl.BlockSpec((1, 1, _CROP, _CROP), lambda b: (b, 0, 0, 0)),
        ],
        out_specs=pl.BlockSpec((1, 1, 128), lambda b: (b, 0, 0)),
        out_shape=jax.ShapeDtypeStruct((_B, 1, 128), jnp.float32),
        compiler_params=pltpu.CompilerParams(
            dimension_semantics=("arbitrary",),
            vmem_limit_bytes=64 * 1024 * 1024,
        ),
    )(gen_output, p_next_step)
    return jnp.sum(out[:, 0, 0])
